# baseline (speedup 1.0000x reference)
import functools

import jax
import jax.numpy as jnp
from jax import lax
from jax.experimental import pallas as pl
from jax.experimental.pallas import tpu as pltpu
from jax.experimental.pallas import tpu_sc as plsc

NC = 2
NS = 16
L = 16
CH = 128
NBUF = 4
SHIFT = 15


def _gate_sc_build(N, NP, HALF, EP):
    per_tile = EP // NS
    n_chunks = per_tile // CH
    acc_stripe = NP // NS
    n_zero = acc_stripe // CH

    mesh = plsc.VectorSubcoreMesh(
        core_axis_name="c", subcore_axis_name="s",
        num_cores=NC, num_subcores=NS)

    @functools.partial(
        pl.kernel,
        out_type=jax.ShapeDtypeStruct((NC, NP, HALF), jnp.float32),
        mesh=mesh,
        compiler_params=pltpu.CompilerParams(use_tc_tiling_on_sc=False),
        scratch_types=[
            pltpu.VMEM_SHARED((NP, HALF), jnp.float32),
            pltpu.VMEM((n_chunks, CH), jnp.int32),
            pltpu.VMEM((NBUF, CH), jnp.int32),
            pltpu.VMEM((NBUF, CH), jnp.int32),
            [pltpu.VMEM((CH, HALF), jnp.float32) for _ in range(NBUF)],
            [pltpu.SemaphoreType.DMA for _ in range(NBUF)],
        ],
    )
    def gate_sc(x2_hbm, enc_hbm, out_hbm,
                acc_sh, enc_i, src_i, dst_i, rows, sems):
        c = lax.axis_index("c")
        s = lax.axis_index("s")

        pltpu.sync_copy(enc_hbm.at[s], enc_i)

        def zbody(i, carry):
            for j in range(HALF // L):
                rows[0][i, pl.ds(j * L, L)] = jnp.zeros((L,), jnp.float32)
            return carry
        lax.fori_loop(0, CH, zbody, 0)
        for k in range(n_zero):
            pltpu.sync_copy(rows[0], acc_sh.at[pl.ds(s * acc_stripe + k * CH, CH)])
        plsc.subcore_barrier()

        def decode(g, slot):
            for j in range(CH // L):
                sl = pl.ds(j * L, L)
                ej = enc_i[g, sl]
                src_i[slot, sl] = jnp.bitwise_and(ej, (1 << SHIFT) - 1) + c
                dst_i[slot, sl] = lax.shift_right_logical(ej, SHIFT)

        def gstart(g, slot):
            pltpu.async_copy(x2_hbm.at[src_i.at[slot]], rows[slot],
                             sems[slot])

        def gwait(slot):
            pltpu.make_async_copy(x2_hbm.at[pl.ds(0, CH)], rows[slot],
                                  sems[slot]).wait()

        for b in range(NBUF - 1):
            decode(b, b)
            gstart(b, b)

        def ebody(i, carry):
            g = i * NBUF
            for b in range(NBUF):
                gb = g + b
                slot_n = (b + NBUF - 1) % NBUF

                @pl.when(gb + NBUF - 1 < n_chunks)
                def _():
                    decode(gb + NBUF - 1, slot_n)
                    gstart(gb + NBUF - 1, slot_n)
                gwait(b)
                pltpu.sync_copy(rows[b], acc_sh.at[dst_i.at[b]], add=True)
            return carry
        lax.fori_loop(0, n_chunks // NBUF, ebody, 0)
        plsc.subcore_barrier()

        r0 = s * acc_stripe
        pltpu.sync_copy(acc_sh.at[pl.ds(r0, acc_stripe)],
                        out_hbm.at[c, pl.ds(r0, acc_stripe)])

    return gate_sc


def _finish_tc(x, parts):
    N, D = x.shape
    BR = 2000

    def body(xb, pb, ob):
        acc = jnp.concatenate([pb[0], pb[1]], axis=-1)
        ob[...] = jnp.maximum(xb[...] * 0.5 + 0.25 * acc, 0.0)

    return pl.pallas_call(
        body,
        grid=(N // BR,),
        in_specs=[pl.BlockSpec((BR, D), lambda i: (i, 0)),
                  pl.BlockSpec((2, BR, D // 2), lambda i: (0, i, 0))],
        out_specs=pl.BlockSpec((BR, D), lambda i: (i, 0)),
        out_shape=jax.ShapeDtypeStruct((N, D), jnp.float32),
    )(x, parts)


def kernel(x, edge_index, edge_weights, w_f_w, w_f_b, w_b_w, w_b_b,
           att_f, att_b):
    N, in_dim = x.shape
    half = in_dim // NC
    E = edge_index.shape[1]

    row = edge_index[0]
    col = edge_index[1]
    chunk_all = NS * CH * NBUF
    EP = ((2 * E + chunk_all - 1) // chunk_all) * chunk_all
    pad = EP - 2 * E
    per_tile = EP // NS
    NP = ((N + 1 + NS * CH - 1) // (NS * CH)) * (NS * CH)

    src = jnp.concatenate([row, col, jnp.zeros((pad,), jnp.int32)])
    dst = jnp.concatenate([col, row, jnp.zeros((pad,), jnp.int32)])
    dst = jnp.where(src == dst, N, dst).astype(jnp.int32)
    enc = ((dst << SHIFT) | (src << 1)).reshape(NS, per_tile // CH, CH)

    x2 = x.reshape(NC * N, half)
    parts = _gate_sc_build(N, NP, half, EP)(x2, enc)
    return _finish_tc(x, parts)


# speedup vs baseline: 1.2658x; 1.2658x over previous
import functools

import jax
import jax.numpy as jnp
from jax import lax
from jax.experimental import pallas as pl
from jax.experimental.pallas import tpu as pltpu
from jax.experimental.pallas import tpu_sc as plsc

NC = 2
NS = 16
L = 16
CH = 128
NBUF = 4
SHIFT = 14


def _gate_sc_build(N, NP, HALF, EP):
    per_tile = EP // NS
    n_chunks = per_tile // CH
    acc_stripe = NP // NS
    n_zero = acc_stripe // CH

    mesh = plsc.VectorSubcoreMesh(
        core_axis_name="c", subcore_axis_name="s",
        num_cores=NC, num_subcores=NS)

    @functools.partial(
        pl.kernel,
        out_type=jax.ShapeDtypeStruct((NC, NP, HALF), jnp.float32),
        mesh=mesh,
        compiler_params=pltpu.CompilerParams(use_tc_tiling_on_sc=False),
        scratch_types=[
            pltpu.VMEM_SHARED((NP, HALF), jnp.float32),
            pltpu.VMEM((n_chunks, CH), jnp.int32),
            pltpu.VMEM((NBUF, CH), jnp.int32),
            pltpu.VMEM((NBUF, CH), jnp.int32),
            [pltpu.VMEM((CH, HALF), jnp.float32) for _ in range(NBUF)],
            pltpu.VMEM((CH, HALF), jnp.float32),
            pltpu.VMEM((CH, HALF), jnp.float32),
            [pltpu.SemaphoreType.DMA for _ in range(NBUF)],
            pltpu.SemaphoreType.DMA,
        ],
    )
    def gate_sc(xcat_hbm, enc_hbm, out_hbm,
                acc_sh, enc_i, src_i, dst_i, rows, xb_v, ab_v,
                sems, sem_o):
        c = lax.axis_index("c")
        s = lax.axis_index("s")
        coff = c * NP

        pltpu.sync_copy(enc_hbm.at[s], enc_i)
        pltpu.sync_copy(xcat_hbm.at[pl.ds(2 * NP - CH, CH)], xb_v)
        for k in range(n_zero):
            pltpu.sync_copy(xb_v, acc_sh.at[pl.ds(s * acc_stripe + k * CH, CH)])
        plsc.subcore_barrier()

        def decode(g, slot):
            for j in range(CH // L):
                sl = pl.ds(j * L, L)
                ej = enc_i[g, sl]
                src_i[slot, sl] = jnp.bitwise_and(ej, (1 << SHIFT) - 1) + coff
                dst_i[slot, sl] = lax.shift_right_logical(ej, SHIFT)

        def gstart(g, slot):
            pltpu.async_copy(xcat_hbm.at[src_i.at[slot]], rows[slot],
                             sems[slot])

        def gwait(slot):
            pltpu.make_async_copy(xcat_hbm.at[pl.ds(0, CH)], rows[slot],
                                  sems[slot]).wait()

        for b in range(NBUF - 1):
            decode(b, b)
            gstart(b, b)

        def ebody(i, carry):
            g = i * NBUF
            for b in range(NBUF):
                gb = g + b
                slot_n = (b + NBUF - 1) % NBUF

                @pl.when(gb + NBUF - 1 < n_chunks)
                def _():
                    decode(gb + NBUF - 1, slot_n)
                    gstart(gb + NBUF - 1, slot_n)
                gwait(b)
                pltpu.sync_copy(rows[b], acc_sh.at[dst_i.at[b]], add=True)
            return carry
        lax.fori_loop(0, n_chunks // NBUF, ebody, 0)
        plsc.subcore_barrier()

        r0 = s * acc_stripe
        pltpu.sync_copy(acc_sh.at[pl.ds(r0, acc_stripe)],
                        out_hbm.at[c, pl.ds(r0, acc_stripe)])

    return gate_sc


def _finish_tc(x, parts):
    N, D = x.shape
    BR = 2000

    def body(xb, pb, ob):
        acc = jnp.concatenate([pb[0], pb[1]], axis=-1)
        ob[...] = jnp.maximum(xb[...] * 0.5 + 0.25 * acc, 0.0)

    return pl.pallas_call(
        body,
        grid=(N // BR,),
        in_specs=[pl.BlockSpec((BR, D), lambda i: (i, 0)),
                  pl.BlockSpec((2, BR, D // 2), lambda i: (0, i, 0))],
        out_specs=pl.BlockSpec((BR, D), lambda i: (i, 0)),
        out_shape=jax.ShapeDtypeStruct((N, D), jnp.float32),
    )(x, parts)


def kernel(x, edge_index, edge_weights, w_f_w, w_f_b, w_b_w, w_b_b,
           att_f, att_b):
    N, in_dim = x.shape
    half = in_dim // NC
    E = edge_index.shape[1]

    row = edge_index[0]
    col = edge_index[1]
    chunk_all = NS * CH * NBUF
    EP = ((2 * E + chunk_all - 1) // chunk_all) * chunk_all
    pad = EP - 2 * E
    per_tile = EP // NS
    NP = ((N + 1 + NS * CH - 1) // (NS * CH)) * (NS * CH)

    src = jnp.concatenate([row, col, jnp.zeros((pad,), jnp.int32)])
    dst = jnp.concatenate([col, row, jnp.zeros((pad,), jnp.int32)])
    dst = jnp.where(src == dst, N, dst).astype(jnp.int32)
    enc = ((dst << SHIFT) | src).reshape(NS, per_tile // CH, CH)

    xh = x.reshape(N, NC, half).transpose(1, 0, 2)
    xcat = jnp.zeros((NC, NP, half), x.dtype).at[:, :N].set(xh)
    xcat = xcat.reshape(NC * NP, half)

    parts = _gate_sc_build(N, NP, half, EP)(xcat, enc)
    return _finish_tc(x, parts)
